# baseline (speedup 1.0000x reference)
import functools

import jax
import jax.numpy as jnp
from jax import lax
from jax.experimental import pallas as pl
from jax.experimental.pallas import tpu as pltpu
from jax.experimental.pallas import tpu_sc as plsc

B, NAT, NNEIGH = 32, 4096, 64
NSPLIT = 2
BH = B // NSPLIT
TPB = 32 // BH
RPT = NAT // TPB
ROWS = 128
NCHUNK = RPT // ROWS

_info = plsc.get_sparse_core_info()
NC, NS = _info.num_cores, _info.num_subcores

_mesh = plsc.VectorSubcoreMesh(core_axis_name="c", subcore_axis_name="s")


@functools.partial(
    pl.kernel,
    out_type=jax.ShapeDtypeStruct((BH, NAT, NNEIGH), jnp.float32),
    mesh=_mesh,
    scratch_types=[
        pltpu.VMEM((NAT,), jnp.float32),
        pltpu.VMEM((2, ROWS, NNEIGH), jnp.int32),
        pltpu.VMEM((2, ROWS, NNEIGH), jnp.float32),
        pltpu.SemaphoreType.DMA,
        pltpu.SemaphoreType.DMA,
        pltpu.SemaphoreType.DMA,
        pltpu.SemaphoreType.DMA,
    ],
    compiler_params=pltpu.CompilerParams(needs_layout_passes=False),
)
def _sc_gather(tab_hbm, idx_hbm, out_hbm, tab_v, idx_v, out_v,
               in_sem0, in_sem1, out_sem0, out_sem1):
    wid = lax.axis_index("s") * NC + lax.axis_index("c")
    bat = wid // TPB
    row0 = (wid % TPB) * RPT
    in_sems = (in_sem0, in_sem1)
    out_sems = (out_sem0, out_sem1)
    pltpu.sync_copy(tab_hbm.at[bat], tab_v)

    in_copies = [None] * NCHUNK
    out_copies = [None] * NCHUNK
    in_copies[0] = pltpu.async_copy(
        idx_hbm.at[bat, pl.ds(row0, ROWS), :], idx_v.at[0], in_sems[0])
    for c in range(NCHUNK):
        buf = c % 2
        if c + 1 < NCHUNK:
            nbuf = (c + 1) % 2
            in_copies[c + 1] = pltpu.async_copy(
                idx_hbm.at[bat, pl.ds(row0 + (c + 1) * ROWS, ROWS), :],
                idx_v.at[nbuf], in_sems[nbuf])
        in_copies[c].wait()
        if c >= 2:
            out_copies[c - 2].wait()

        @plsc.parallel_loop(0, ROWS, step=1, unroll=8)
        def _body(r):
            for j in range(0, NNEIGH, 16):
                ids = idx_v[buf, r, pl.ds(j, 16)]
                out_v[buf, r, pl.ds(j, 16)] = plsc.load_gather(tab_v, [ids])

        out_copies[c] = pltpu.async_copy(
            out_v.at[buf],
            out_hbm.at[bat, pl.ds(row0 + c * ROWS, ROWS), :],
            out_sems[buf])
    out_copies[NCHUNK - 2].wait()
    out_copies[NCHUNK - 1].wait()


def kernel(atomic_numbers, neighbors):
    tab = atomic_numbers.reshape(B, NAT)
    parts = [
        _sc_gather(tab[i * BH:(i + 1) * BH], neighbors[i * BH:(i + 1) * BH])
        for i in range(NSPLIT)
    ]
    return jnp.concatenate(parts, axis=0)[..., None]


# speedup vs baseline: 1.4597x; 1.4597x over previous
import functools

import jax
import jax.numpy as jnp
from jax import lax
from jax.experimental import pallas as pl
from jax.experimental.pallas import tpu as pltpu
from jax.experimental.pallas import tpu_sc as plsc

B, NAT, NNEIGH = 32, 4096, 64
ROWS = 128
NCHUNK = NAT // ROWS
NBUF = 3

_info = plsc.get_sparse_core_info()
NC, NS = _info.num_cores, _info.num_subcores

_mesh = plsc.VectorSubcoreMesh(core_axis_name="c", subcore_axis_name="s")


@functools.partial(
    pl.kernel,
    out_type=jax.ShapeDtypeStruct((B, NAT, NNEIGH), jnp.float32),
    mesh=_mesh,
    scratch_types=[
        pltpu.VMEM((NAT,), jnp.float32),
        pltpu.VMEM((NBUF, ROWS, NNEIGH), jnp.int32),
        pltpu.VMEM((NBUF, ROWS, NNEIGH), jnp.float32),
    ]
    + [pltpu.SemaphoreType.DMA] * (2 * NBUF),
    compiler_params=pltpu.CompilerParams(needs_layout_passes=False),
)
def _sc_gather(tab_hbm, idx_hbm, out_hbm, tab_v, idx_v, out_v, *sems):
    wid = lax.axis_index("s") * NC + lax.axis_index("c")
    in_sems = sems[:NBUF]
    out_sems = sems[NBUF:]
    pltpu.sync_copy(tab_hbm.at[wid], tab_v)

    in_copies = [None] * NCHUNK
    out_copies = [None] * NCHUNK
    for p in range(NBUF - 1):
        in_copies[p] = pltpu.async_copy(
            idx_hbm.at[wid, pl.ds(p * ROWS, ROWS), :], idx_v.at[p],
            in_sems[p])
    for c in range(NCHUNK):
        buf = c % NBUF
        if c + NBUF - 1 < NCHUNK:
            nxt = c + NBUF - 1
            nbuf = nxt % NBUF
            in_copies[nxt] = pltpu.async_copy(
                idx_hbm.at[wid, pl.ds(nxt * ROWS, ROWS), :],
                idx_v.at[nbuf], in_sems[nbuf])
        in_copies[c].wait()
        if c >= NBUF:
            out_copies[c - NBUF].wait()

        @plsc.parallel_loop(0, ROWS, step=1, unroll=8)
        def _body(r):
            for j in range(0, NNEIGH, 16):
                ids = idx_v[buf, r, pl.ds(j, 16)]
                out_v[buf, r, pl.ds(j, 16)] = plsc.load_gather(tab_v, [ids])

        out_copies[c] = pltpu.async_copy(
            out_v.at[buf],
            out_hbm.at[wid, pl.ds(c * ROWS, ROWS), :],
            out_sems[buf])
    for c in range(max(0, NCHUNK - NBUF), NCHUNK):
        out_copies[c].wait()


def kernel(atomic_numbers, neighbors):
    tab = atomic_numbers.reshape(B, NAT)
    return _sc_gather(tab, neighbors)[..., None]
